# baseline (speedup 1.0000x reference)
import jax
import jax.numpy as jnp
from jax.experimental import pallas as pl
from jax.experimental.pallas import tpu as pltpu

_Q, _K, _D = 1024, 100000, 16
_TK = 102400
_TH = 5120
_NH = _TK // _TH
_LANES = 128
_CHUNKS = _TH // _LANES
_KPAD = ((_K + _TK - 1) // _TK) * _TK
_NSTEPS = _KPAD // _TK
_INT_MAX = 2**31 - 1


def _nn_body(q_ref, kt_ref, k2_ref, q2_ref,
             scores_ref, idx_ref, img_ref,
             accv_ref, accs_ref):
    i = pl.program_id(0)

    @pl.when(i == 0)
    def _init():
        accv_ref[...] = jnp.full((_Q, _LANES), jnp.inf, jnp.float32)
        accs_ref[...] = jnp.zeros((_Q, _LANES), jnp.int32)

    q = q_ref[...]
    kt = kt_ref[...]
    k2 = k2_ref[...]
    q2 = q2_ref[...]

    best_v = None
    best_c = None
    for h in range(_NH):
        kth = kt[:, h * _TH:(h + 1) * _TH]
        k2h = k2[:, h * _TH:(h + 1) * _TH]
        dots = jax.lax.dot_general(
            q, kth, (((1,), (0,)), ((), ())),
            preferred_element_type=jnp.float32)
        t = q2 + k2h
        d2 = t + dots
        hv = d2[:, 0:_LANES]
        hc = jnp.zeros((_Q, _LANES), jnp.int32)
        for c in range(1, _CHUNKS):
            cand = d2[:, c * _LANES:(c + 1) * _LANES]
            lt = cand < hv
            hv = jnp.where(lt, cand, hv)
            hc = jnp.where(lt, c, hc)
        hc = hc + h * _CHUNKS
        if best_v is None:
            best_v, best_c = hv, hc
        else:
            lt = hv < best_v
            best_v = jnp.where(lt, hv, best_v)
            best_c = jnp.where(lt, hc, best_c)
    av = accv_ref[...]
    ast = accs_ref[...]
    lt = best_v < av
    accv_ref[...] = jnp.where(lt, best_v, av)
    accs_ref[...] = jnp.where(lt, i * (_NH * _CHUNKS) + best_c, ast)

    @pl.when(i == _NSTEPS - 1)
    def _finalize():
        vals = accv_ref[...]
        steps = accs_ref[...]
        lane = jax.lax.broadcasted_iota(jnp.int32, (_Q, _LANES), 1)
        gidx = steps * _LANES + lane
        dist = jnp.sqrt(jnp.maximum(vals, 0.0) + 1e-12)
        best = jnp.min(dist, axis=1, keepdims=True)
        tie = dist == best
        cand_idx = jnp.where(tie, gidx, jnp.full_like(gidx, _INT_MAX))
        best_idx = jnp.min(cand_idx, axis=1, keepdims=True)
        scores_ref[...] = best
        idx_ref[...] = best_idx
        img_ref[...] = jnp.max(best, axis=0, keepdims=True)


def kernel(queries, keys):
    q2 = jnp.sum(queries * queries, axis=1, keepdims=True)
    k2 = jnp.sum(keys * keys, axis=1)
    qs = -2.0 * queries
    kt = keys.T
    k2p = jnp.pad(k2, (0, _KPAD - _K), constant_values=jnp.nan)
    k2p = k2p.reshape(1, _KPAD)

    scores, idx, img = pl.pallas_call(
        _nn_body,
        grid=(_NSTEPS,),
        in_specs=[
            pl.BlockSpec((_Q, _D), lambda i: (0, 0)),
            pl.BlockSpec((_D, _TK), lambda i: (0, i)),
            pl.BlockSpec((1, _TK), lambda i: (0, i)),
            pl.BlockSpec((_Q, 1), lambda i: (0, 0)),
        ],
        out_specs=[
            pl.BlockSpec((_Q, 1), lambda i: (0, 0)),
            pl.BlockSpec((_Q, 1), lambda i: (0, 0)),
            pl.BlockSpec((1, 1), lambda i: (0, 0)),
        ],
        out_shape=[
            jax.ShapeDtypeStruct((_Q, 1), jnp.float32),
            jax.ShapeDtypeStruct((_Q, 1), jnp.int32),
            jax.ShapeDtypeStruct((1, 1), jnp.float32),
        ],
        scratch_shapes=[
            pltpu.VMEM((_Q, _LANES), jnp.float32),
            pltpu.VMEM((_Q, _LANES), jnp.int32),
        ],
    )(qs, kt, k2p, q2)
    return scores[:, 0], idx[:, 0], img[0, 0]


# speedup vs baseline: 1.0324x; 1.0324x over previous
import jax
import jax.numpy as jnp
from jax.experimental import pallas as pl
from jax.experimental.pallas import tpu as pltpu

_Q, _K, _D = 1024, 100000, 16
_TK = 25600
_TH = 5120
_NH = _TK // _TH
_LANES = 128
_CHUNKS = _TH // _LANES
_KPAD = ((_K + _TK - 1) // _TK) * _TK
_NSTEPS = _KPAD // _TK
_INT_MAX = 2**31 - 1


def _nn_body(q_ref, kt_ref, k2_ref, q2_ref,
             scores_ref, idx_ref, img_ref,
             accv_ref, accs_ref):
    i = pl.program_id(0)

    @pl.when(i == 0)
    def _init():
        accv_ref[...] = jnp.full((_Q, _LANES), jnp.inf, jnp.float32)
        accs_ref[...] = jnp.zeros((_Q, _LANES), jnp.int32)

    q = q_ref[...]
    kt = kt_ref[...]
    k2 = k2_ref[...]
    q2 = q2_ref[...]

    best_v = None
    best_c = None
    for h in range(_NH):
        kth = kt[:, h * _TH:(h + 1) * _TH]
        k2h = k2[:, h * _TH:(h + 1) * _TH]
        dots = jax.lax.dot_general(
            q, kth, (((1,), (0,)), ((), ())),
            preferred_element_type=jnp.float32)
        t = q2 + k2h
        d2 = t + dots
        hv = d2[:, 0:_LANES]
        hc = jnp.zeros((_Q, _LANES), jnp.int32)
        for c in range(1, _CHUNKS):
            cand = d2[:, c * _LANES:(c + 1) * _LANES]
            lt = cand < hv
            hv = jnp.where(lt, cand, hv)
            hc = jnp.where(lt, c, hc)
        hc = hc + h * _CHUNKS
        if best_v is None:
            best_v, best_c = hv, hc
        else:
            lt = hv < best_v
            best_v = jnp.where(lt, hv, best_v)
            best_c = jnp.where(lt, hc, best_c)
    av = accv_ref[...]
    ast = accs_ref[...]
    lt = best_v < av
    accv_ref[...] = jnp.where(lt, best_v, av)
    accs_ref[...] = jnp.where(lt, i * (_NH * _CHUNKS) + best_c, ast)

    @pl.when(i == _NSTEPS - 1)
    def _finalize():
        vals = accv_ref[...]
        steps = accs_ref[...]
        lane = jax.lax.broadcasted_iota(jnp.int32, (_Q, _LANES), 1)
        gidx = steps * _LANES + lane
        dist = jnp.sqrt(jnp.maximum(vals, 0.0) + 1e-12)
        best = jnp.min(dist, axis=1, keepdims=True)
        tie = dist == best
        cand_idx = jnp.where(tie, gidx, jnp.full_like(gidx, _INT_MAX))
        best_idx = jnp.min(cand_idx, axis=1, keepdims=True)
        scores_ref[...] = best
        idx_ref[...] = best_idx
        img_ref[...] = jnp.max(best, axis=0, keepdims=True)


def kernel(queries, keys):
    q2 = jnp.sum(queries * queries, axis=1, keepdims=True)
    k2 = jnp.sum(keys * keys, axis=1)
    qs = -2.0 * queries
    kt = keys.T
    k2p = jnp.pad(k2, (0, _KPAD - _K), constant_values=jnp.nan)
    k2p = k2p.reshape(1, _KPAD)

    scores, idx, img = pl.pallas_call(
        _nn_body,
        grid=(_NSTEPS,),
        in_specs=[
            pl.BlockSpec((_Q, _D), lambda i: (0, 0)),
            pl.BlockSpec((_D, _TK), lambda i: (0, i)),
            pl.BlockSpec((1, _TK), lambda i: (0, i)),
            pl.BlockSpec((_Q, 1), lambda i: (0, 0)),
        ],
        out_specs=[
            pl.BlockSpec((_Q, 1), lambda i: (0, 0)),
            pl.BlockSpec((_Q, 1), lambda i: (0, 0)),
            pl.BlockSpec((1, 1), lambda i: (0, 0)),
        ],
        out_shape=[
            jax.ShapeDtypeStruct((_Q, 1), jnp.float32),
            jax.ShapeDtypeStruct((_Q, 1), jnp.int32),
            jax.ShapeDtypeStruct((1, 1), jnp.float32),
        ],
        scratch_shapes=[
            pltpu.VMEM((_Q, _LANES), jnp.float32),
            pltpu.VMEM((_Q, _LANES), jnp.int32),
        ],
    )(qs, kt, k2p, q2)
    return scores[:, 0], idx[:, 0], img[0, 0]
